# Initial kernel scaffold; baseline (speedup 1.0000x reference)
import jax
import jax.numpy as jnp
from jax.experimental import pallas as pl


def kernel(student_logits, batch_indices, teacher_indices, teacher_scores):
    raise NotImplementedError("write your pallas kernel here")


# speedup vs baseline: 11.9945x; 11.9945x over previous
import functools

import jax
import jax.numpy as jnp
from jax import lax
from jax.experimental import pallas as pl
from jax.experimental.pallas import tpu as pltpu
from jax.experimental.pallas import tpu_sc as plsc

_B = 4096
_K = 50
_T = 2.0
_VP = 1 << 20
_NC, _NS = 2, 16
_NW = _NC * _NS
_CH = _VP // _NW
_E = (_B * _K) // _NW
_CHUNK = 128
_GRP = 10

def _build_table(bidx_hbm, table_hbm, buf_v, bidx_v):
    wid = lax.axis_index("s") * _NC + lax.axis_index("c")
    base = pl.multiple_of(wid * _CH, _CH)
    neg1 = jnp.full((16,), -1, jnp.int32)

    def memset(i, c):
        for b in range(8):
            buf_v[pl.ds((i * 8 + b) * 16, 16)] = neg1
        return c
    lax.fori_loop(0, _CH // 128, memset, 0)

    pltpu.sync_copy(bidx_hbm, bidx_v)
    lane = lax.iota(jnp.int32, 16)

    def scat(i, c):
        g = bidx_v[pl.ds(i * 16, 16)]
        m = (g >= base) & (g < base + _CH)
        plsc.store_scatter(buf_v, [g - base], lane + i * 16, mask=m)
        return c
    lax.fori_loop(0, _B // 16, scat, 0)

    pltpu.sync_copy(buf_v, table_hbm.at[pl.ds(base, _CH)])


def _gather_pairs(table_hbm, tidx_hbm, slog_hbm, pos_hbm, sval_hbm,
                  tidx_v, pos_v, flat_v, sval_v, sem):
    wid = lax.axis_index("s") * _NC + lax.axis_index("c")
    base = pl.multiple_of(wid * _E, 8)
    pltpu.sync_copy(tidx_hbm.at[pl.ds(base, _E)], tidx_v)

    def table_grp(gi, c):
        descs = []
        for b in range(_GRP):
            off = pl.multiple_of((gi * _GRP + b) * _CHUNK, _CHUNK)
            descs.append(pltpu.async_copy(
                table_hbm.at[tidx_v.at[pl.ds(off, _CHUNK)]],
                pos_v.at[pl.ds(off, _CHUNK)], sem))
        for d in descs:
            d.wait()
        return c
    lax.fori_loop(0, _E // (_CHUNK * _GRP), table_grp, 0)

    lane = lax.iota(jnp.int32, 16)

    def mkflat(i, c):
        e = base + i * 16 + lane
        row = e // _K
        p = pos_v[pl.ds(i * 16, 16)]
        flat_v[pl.ds(i * 16, 16)] = row * _B + jnp.maximum(p, 0)
        return c
    lax.fori_loop(0, _E // 16, mkflat, 0)

    def val_grp(gi, c):
        descs = []
        for b in range(_GRP):
            off = pl.multiple_of((gi * _GRP + b) * _CHUNK, _CHUNK)
            descs.append(pltpu.async_copy(
                slog_hbm.at[flat_v.at[pl.ds(off, _CHUNK)]],
                sval_v.at[pl.ds(off, _CHUNK)], sem))
        for d in descs:
            d.wait()
        return c
    lax.fori_loop(0, _E // (_CHUNK * _GRP), val_grp, 0)

    pltpu.sync_copy(pos_v, pos_hbm.at[pl.ds(base, _E)])
    pltpu.sync_copy(sval_v, sval_hbm.at[pl.ds(base, _E)])


_R = 256


def _lse_body(x_ref, lse_ref, diag_ref):
    i = pl.program_id(0)
    x = x_ref[...]
    xs = x * (1.0 / _T)
    m = jnp.max(xs, axis=1, keepdims=True)
    s = jnp.sum(jnp.exp(xs - m), axis=1)
    lse_ref[0, 0, :] = m[:, 0] + jnp.log(s)
    rows = i * _R + lax.broadcasted_iota(jnp.int32, (_R, _B), 0)
    cols = lax.broadcasted_iota(jnp.int32, (_R, _B), 1)
    diag_ref[0, 0, :] = jnp.sum(jnp.where(rows == cols, x, 0.0), axis=1)


def _loss_body(pos_ref, sc_ref, sv_ref, lse_ref, dg_ref, out_ref):
    pos = pos_ref[...]
    sc = sc_ref[...]
    sv = sv_ref[...]
    lse = lse_ref[...]
    dg = dg_ref[...]
    rows = lax.broadcasted_iota(jnp.int32, (_B, _K), 0)
    offd = (pos >= 0) & (pos != rows)
    w = jnp.where(offd, sc, 0.0)
    rs = 1.0 + jnp.sum(w, axis=1, keepdims=True)
    live = offd & (sc > 0)
    t_safe = jnp.where(live, sc, 1.0) / rs
    logp = sv * (1.0 / _T) - lse
    term = jnp.where(live, (w / rs) * (jnp.log(t_safe) - logp), 0.0)
    tii = 1.0 / rs
    term_ii = tii * (jnp.log(tii) - (dg * (1.0 / _T) - lse))
    total = jnp.sum(term) + jnp.sum(term_ii)
    out_ref[...] = jnp.full((1, 1), total * (_T * _T / _B), jnp.float32)


@functools.lru_cache(maxsize=1)
def _sc_kernels():
    mesh = plsc.VectorSubcoreMesh(core_axis_name="c", subcore_axis_name="s",
                                  num_cores=_NC, num_subcores=_NS)
    params = pltpu.CompilerParams(needs_layout_passes=False)
    build_table = pl.kernel(
        _build_table, mesh=mesh, compiler_params=params,
        out_type=jax.ShapeDtypeStruct((_VP,), jnp.int32),
        scratch_types=[pltpu.VMEM((_CH,), jnp.int32),
                       pltpu.VMEM((_B,), jnp.int32)],
    )
    gather_pairs = pl.kernel(
        _gather_pairs, mesh=mesh, compiler_params=params,
        out_type=(jax.ShapeDtypeStruct((_B * _K,), jnp.int32),
                  jax.ShapeDtypeStruct((_B * _K,), jnp.float32)),
        scratch_types=[pltpu.VMEM((_E,), jnp.int32),
                       pltpu.VMEM((_E,), jnp.int32),
                       pltpu.VMEM((_E,), jnp.int32),
                       pltpu.VMEM((_E,), jnp.float32),
                       pltpu.SemaphoreType.DMA],
    )
    return build_table, gather_pairs


def kernel(student_logits, batch_indices, teacher_indices, teacher_scores):
    build_table, gather_pairs = _sc_kernels()
    bidx = batch_indices.astype(jnp.int32)
    tidx = teacher_indices.astype(jnp.int32).reshape(-1)
    table = build_table(bidx)
    pos_f, sval_f = gather_pairs(table, tidx, student_logits.reshape(-1))

    lse3, dg3 = pl.pallas_call(
        _lse_body,
        grid=(_B // _R,),
        in_specs=[pl.BlockSpec((_R, _B), lambda i: (i, 0))],
        out_specs=[pl.BlockSpec((1, 1, _R), lambda i: (i, 0, 0)),
                   pl.BlockSpec((1, 1, _R), lambda i: (i, 0, 0))],
        out_shape=[jax.ShapeDtypeStruct((_B // _R, 1, _R), jnp.float32),
                   jax.ShapeDtypeStruct((_B // _R, 1, _R), jnp.float32)],
    )(student_logits)

    out = pl.pallas_call(
        _loss_body,
        out_shape=jax.ShapeDtypeStruct((1, 1), jnp.float32),
    )(pos_f.reshape(_B, _K), teacher_scores, sval_f.reshape(_B, _K),
      lse3.reshape(_B, 1), dg3.reshape(_B, 1))
    return out[0, 0]
